# Initial kernel scaffold; baseline (speedup 1.0000x reference)
import jax
import jax.numpy as jnp
from jax.experimental import pallas as pl


def kernel(x, table):
    raise NotImplementedError("write your pallas kernel here")


# speedup vs baseline: 6.3738x; 6.3738x over previous
import functools

import jax
import jax.numpy as jnp
from jax import lax
from jax.experimental import pallas as pl
from jax.experimental.pallas import tpu as pltpu
from jax.experimental.pallas import tpu_sc as plsc

NC, NS, L = 2, 16, 16
NW = NC * NS
D = 128
CHUNK = 128


def _make_kernel(n_rows, vocab):
    rows_per_w = n_rows // NW
    n_chunks = rows_per_w // CHUNK
    mesh = plsc.VectorSubcoreMesh(core_axis_name="c", subcore_axis_name="s")

    @functools.partial(
        pl.kernel,
        out_type=jax.ShapeDtypeStruct((n_rows, D), jnp.float32),
        mesh=mesh,
        scratch_types=[
            pltpu.VMEM((n_chunks, CHUNK), jnp.int32),
            pltpu.VMEM((CHUNK, D), jnp.float32),
            pltpu.SemaphoreType.DMA,
        ],
    )
    def emb_kernel(idx_hbm, table_hbm, out_hbm, idx_v, rows_v, sem):
        wid = lax.axis_index("s") * NC + lax.axis_index("c")
        base = wid * rows_per_w
        pltpu.sync_copy(idx_hbm.at[wid], idx_v)

        @pl.loop(0, n_chunks)
        def _chunk(g):
            pltpu.async_copy(table_hbm.at[idx_v.at[g]], rows_v, sem).wait()
            pltpu.sync_copy(rows_v, out_hbm.at[pl.ds(base + g * CHUNK, CHUNK)])

    return emb_kernel


def kernel(x, table):
    b, s = x.shape
    n_rows = b * s
    idx3 = x.reshape(NW, n_rows // (NW * CHUNK), CHUNK)
    out = _make_kernel(n_rows, table.shape[0])(idx3, table)
    return out.reshape(b, s, D)


# speedup vs baseline: 9.2393x; 1.4496x over previous
import functools

import jax
import jax.numpy as jnp
from jax import lax
from jax.experimental import pallas as pl
from jax.experimental.pallas import tpu as pltpu
from jax.experimental.pallas import tpu_sc as plsc

NC, NS, L = 2, 16, 16
NW = NC * NS
D = 128
CHUNK = 128
NBUF = 4


def _make_kernel(n_rows):
    rows_per_w = n_rows // NW
    n_chunks = rows_per_w // CHUNK
    n_epochs = n_chunks // NBUF
    mesh = plsc.VectorSubcoreMesh(core_axis_name="c", subcore_axis_name="s")

    @functools.partial(
        pl.kernel,
        out_type=jax.ShapeDtypeStruct((n_rows, D), jnp.float32),
        mesh=mesh,
        scratch_types=[
            pltpu.VMEM((n_chunks, CHUNK), jnp.int32),
            [pltpu.VMEM((CHUNK, D), jnp.float32)] * NBUF,
            [pltpu.SemaphoreType.DMA] * NBUF,
            [pltpu.SemaphoreType.DMA] * NBUF,
        ],
    )
    def emb_kernel(idx_hbm, table_hbm, out_hbm, idx_v, bufs, gsems, ssems):
        wid = lax.axis_index("s") * NC + lax.axis_index("c")
        base = wid * rows_per_w
        pltpu.sync_copy(idx_hbm.at[wid], idx_v)

        def gather(c, b):
            return pltpu.make_async_copy(
                table_hbm.at[idx_v.at[c]], bufs[b], gsems[b])

        def store(c, b):
            return pltpu.make_async_copy(
                bufs[b], out_hbm.at[pl.ds(base + c * CHUNK, CHUNK)], ssems[b])

        for b in range(NBUF):
            gather(b, b).start()

        @pl.loop(0, n_epochs - 1)
        def _epoch(t):
            for b in range(NBUF):
                c = t * NBUF + b
                gather(c, b).wait()
                store(c, b).start()
                store(c, b).wait()
                gather(c + NBUF, b).start()

        for b in range(NBUF):
            c = n_chunks - NBUF + b
            gather(c, b).wait()
            store(c, b).start()
        for b in range(NBUF):
            store(n_chunks - NBUF + b, b).wait()

    return emb_kernel


def kernel(x, table):
    b, s = x.shape
    n_rows = b * s
    idx3 = x.reshape(NW, n_rows // (NW * CHUNK), CHUNK)
    out = _make_kernel(n_rows)(idx3, table)
    return out.reshape(b, s, D)


# speedup vs baseline: 9.2593x; 1.0022x over previous
import functools

import jax
import jax.numpy as jnp
from jax import lax
from jax.experimental import pallas as pl
from jax.experimental.pallas import tpu as pltpu
from jax.experimental.pallas import tpu_sc as plsc

NC, NS, L = 2, 16, 16
NW = NC * NS
D = 128
CHUNK = 128
NBUF = 5


def _make_kernel(n_rows):
    rows_per_w = n_rows // NW
    n_chunks = rows_per_w // CHUNK
    n_epochs = n_chunks // NBUF
    mesh = plsc.VectorSubcoreMesh(core_axis_name="c", subcore_axis_name="s")

    @functools.partial(
        pl.kernel,
        out_type=jax.ShapeDtypeStruct((n_rows, D), jnp.float32),
        mesh=mesh,
        scratch_types=[
            pltpu.VMEM((n_chunks, CHUNK), jnp.int32),
            [pltpu.VMEM((CHUNK, D), jnp.float32)] * NBUF,
            [pltpu.SemaphoreType.DMA] * NBUF,
            [pltpu.SemaphoreType.DMA] * NBUF,
        ],
    )
    def emb_kernel(idx_hbm, table_hbm, out_hbm, idx_v, bufs, gsems, ssems):
        wid = lax.axis_index("s") * NC + lax.axis_index("c")
        base = wid * rows_per_w
        pltpu.sync_copy(idx_hbm.at[wid], idx_v)

        def gather(c, b):
            return pltpu.make_async_copy(
                table_hbm.at[idx_v.at[c]], bufs[b], gsems[b])

        def store(c, b):
            return pltpu.make_async_copy(
                bufs[b], out_hbm.at[pl.ds(base + c * CHUNK, CHUNK)], ssems[b])

        for b in range(NBUF):
            gather(b, b).start()

        @pl.loop(0, n_epochs - 1)
        def _epoch(t):
            for b in range(NBUF):
                c = t * NBUF + b
                gather(c, b).wait()
                store(c, b).start()
                store(c, b).wait()
                gather(c + NBUF, b).start()

        for b in range(NBUF):
            c = n_chunks - NBUF + b
            gather(c, b).wait()
            store(c, b).start()
        for b in range(NBUF):
            store(n_chunks - NBUF + b, b).wait()

    return emb_kernel


def kernel(x, table):
    b, s = x.shape
    n_rows = b * s
    idx3 = x.reshape(NW, n_rows // (NW * CHUNK), CHUNK)
    out = _make_kernel(n_rows)(idx3, table)
    return out.reshape(b, s, D)
